# baseline (speedup 1.0000x reference)
import functools

import jax
import jax.numpy as jnp
from jax import lax
from jax.experimental import pallas as pl
from jax.experimental.pallas import tpu as pltpu
from jax.experimental.pallas import tpu_sc as plsc

NC = 2
NS = 16
NW = NC * NS


def _sc_body(n_cls, n_ctx, s, d, base_cnt, rem,
             ctx_hbm, pre_hbm, suf_hbm, out_hbm,
             ctx_v, pre_v, suf_v, sem_si, sem_so, sem_co, sem_po):
    cid = lax.axis_index("c")
    sid = lax.axis_index("s")
    wid = sid * NC + cid
    base = wid * base_cnt + jnp.minimum(wid, rem)
    has_extra = wid < rem

    pltpu.sync_copy(ctx_hbm, ctx_v)
    pltpu.sync_copy(pre_hbm.at[pl.ds(base, base_cnt)], pre_v.at[pl.ds(0, base_cnt)])

    @pl.when(has_extra)
    def _():
        pltpu.sync_copy(pre_hbm.at[pl.ds(base + base_cnt, 1)],
                        pre_v.at[pl.ds(base_cnt, 1)])

    def suf_in(i):
        return pltpu.make_async_copy(
            suf_hbm.at[pl.ds(base + i, 1)], suf_v.at[pl.ds(i % 2, 1)], sem_si)

    def suf_out(i):
        return pltpu.make_async_copy(
            suf_v.at[pl.ds(i % 2, 1)],
            out_hbm.at[pl.ds(base + i, 1), pl.ds(1 + n_ctx, s)], sem_so)

    def small_out(i):
        pltpu.make_async_copy(
            ctx_v, out_hbm.at[pl.ds(base + i, 1), pl.ds(1, n_ctx)], sem_co).start()
        pltpu.make_async_copy(
            pre_v.at[pl.ds(i, 1)],
            out_hbm.at[pl.ds(base + i, 1), pl.ds(0, 1)], sem_po).start()

    suf_in(0).start()
    for i in range(base_cnt):
        suf_in(i).wait()
        if i >= 1:
            suf_out(i - 1).wait()
        if i + 1 < base_cnt:
            suf_in(i + 1).start()
        elif i + 1 == base_cnt:
            @pl.when(has_extra)
            def _():
                suf_in(base_cnt).start()
        suf_out(i).start()
        small_out(i)
    suf_out(base_cnt - 1).wait()

    @pl.when(has_extra)
    def _():
        suf_in(base_cnt).wait()
        suf_out(base_cnt).start()
        small_out(base_cnt)
        suf_out(base_cnt).wait()

    for i in range(base_cnt):
        pltpu.make_async_copy(
            ctx_v, out_hbm.at[pl.ds(base, 1), pl.ds(1, n_ctx)], sem_co).wait()
        pltpu.make_async_copy(
            pre_v.at[pl.ds(0, 1)],
            out_hbm.at[pl.ds(base, 1), pl.ds(0, 1)], sem_po).wait()

    @pl.when(has_extra)
    def _():
        pltpu.make_async_copy(
            ctx_v, out_hbm.at[pl.ds(base, 1), pl.ds(1, n_ctx)], sem_co).wait()
        pltpu.make_async_copy(
            pre_v.at[pl.ds(0, 1)],
            out_hbm.at[pl.ds(base, 1), pl.ds(0, 1)], sem_po).wait()


def kernel(ctx, token_prefix, token_suffix):
    n_cls, _, d = token_prefix.shape
    n_ctx = ctx.shape[0]
    s = token_suffix.shape[1]
    seq = 1 + n_ctx + s
    base_cnt = n_cls // NW
    rem = n_cls - base_cnt * NW

    ctx3 = ctx.reshape(1, n_ctx, d)
    mesh = plsc.VectorSubcoreMesh(core_axis_name="c", subcore_axis_name="s")

    sck = functools.partial(
        pl.kernel,
        out_type=jax.ShapeDtypeStruct((n_cls, seq, d), jnp.float32),
        mesh=mesh,
        compiler_params=pltpu.CompilerParams(use_tc_tiling_on_sc=False),
        scratch_types=[
            pltpu.VMEM((1, n_ctx, d), jnp.float32),
            pltpu.VMEM((base_cnt + 1, 1, d), jnp.float32),
            pltpu.VMEM((2, s, d), jnp.float32),
            pltpu.SemaphoreType.DMA,
            pltpu.SemaphoreType.DMA,
            pltpu.SemaphoreType.DMA,
            pltpu.SemaphoreType.DMA,
        ],
    )(functools.partial(_sc_body, n_cls, n_ctx, s, d, base_cnt, rem))

    return sck(ctx3, token_prefix, token_suffix)
